# Initial kernel scaffold; baseline (speedup 1.0000x reference)
import jax
import jax.numpy as jnp
from jax.experimental import pallas as pl


def kernel(x, w1f, shift1, w2f, shift2, fc1_w, fc1_b, fc2_w, fc2_b):
    raise NotImplementedError("write your pallas kernel here")


# speedup vs baseline: 17.0364x; 17.0364x over previous
import jax
import jax.numpy as jnp
from jax.experimental import pallas as pl
from jax.experimental.pallas import tpu as pltpu

_BT = 32


def _toeplitz1(w1f):
    xs = jnp.concatenate([jnp.arange(0, 28, 2), jnp.arange(1, 28, 2)])
    xin = jnp.arange(30)
    d = xin[:, None] - xs[None, :]
    valid = (d >= 0) & (d <= 2)
    kh = jnp.arange(3)[:, None, None]
    tap = kh * 3 + jnp.clip(d, 0, 2)[None]
    t = w1f[tap, 0, :] * valid[None, :, :, None]
    return t.reshape(90, 896)


def _toeplitz2(w2f):
    xs = jnp.concatenate([jnp.arange(0, 14, 2), jnp.arange(1, 14, 2)])
    xin = jnp.arange(16)
    d = xin[:, None] - xs[None, :]
    valid = (d >= 0) & (d <= 2)
    kh = jnp.arange(3)[:, None, None]
    tap = kh * 3 + jnp.clip(d, 0, 2)[None]
    t = w2f[tap] * valid[None, :, :, None, None]
    t = t.transpose(0, 1, 3, 2, 4)
    return t.reshape(1536, 896)


def _fused_cnn_kernel(xp_ref, b1_ref, s1_ref, b2_ref, s2_ref,
                      fw1_ref, fb1_ref, fw2_ref, fb2_ref, o_ref):
    bsz = xp_ref.shape[0]
    xp = xp_ref[...]
    xr = xp.reshape(bsz, 15, 2, 30)
    xe = xr[:, :, 0, :]
    xo = xr[:, :, 1, :]

    lhs_e = jnp.concatenate([xe[:, 0:14], xo[:, 0:14], xe[:, 1:15]], axis=-1)
    lhs_o = jnp.concatenate([xo[:, 0:14], xe[:, 1:15], xo[:, 1:15]], axis=-1)
    lhs1 = jnp.concatenate([lhs_e, lhs_o], axis=1).reshape(bsz * 28, 90)
    c1 = jnp.dot(lhs1, b1_ref[...],
                 preferred_element_type=jnp.float32).reshape(bsz, 28, 896)
    m1 = jnp.maximum(c1[:, :14], c1[:, 14:])
    m1 = jnp.maximum(m1[..., :448], m1[..., 448:])
    p1 = jnp.maximum(m1 + s1_ref[...], 0.0)

    z32 = jnp.zeros((bsz, 14, 32), jnp.float32)
    p1p = jnp.concatenate([z32, p1, z32], axis=-1)
    zrow = jnp.zeros((bsz, 1, 512), jnp.float32)
    p1pp = jnp.concatenate([zrow, p1p, zrow], axis=1)
    pr = p1pp.reshape(bsz, 8, 2, 512)
    re = pr[:, :, 0]
    ro = pr[:, :, 1]

    l2e = jnp.concatenate([re[:, 0:7], ro[:, 0:7], re[:, 1:8]], axis=-1)
    l2o = jnp.concatenate([ro[:, 0:7], re[:, 1:8], ro[:, 1:8]], axis=-1)
    lhs2 = jnp.concatenate([l2e, l2o], axis=1).reshape(bsz * 14, 1536)
    c2 = jnp.dot(lhs2, b2_ref[...],
                 preferred_element_type=jnp.float32).reshape(bsz, 14, 896)
    m2 = jnp.maximum(c2[:, :7], c2[:, 7:])
    m2 = jnp.maximum(m2[..., :448], m2[..., 448:])
    p2 = jnp.maximum(m2 + s2_ref[...], 0.0)

    z64 = jnp.zeros((bsz, 7, 64), jnp.float32)
    flat = jnp.concatenate([p2, z64], axis=-1).reshape(bsz, 3584)
    h1 = jnp.dot(flat, fw1_ref[...],
                 preferred_element_type=jnp.float32) + fb1_ref[...]
    h1 = jnp.maximum(h1, 0.0)
    out = jnp.dot(h1, fw2_ref[...],
                  preferred_element_type=jnp.float32) + fb2_ref[...]
    o_ref[...] = out


def kernel(x, w1f, shift1, w2f, shift2, fc1_w, fc1_b, fc2_w, fc2_b):
    n = x.shape[0]
    xp = jnp.pad(x.reshape(n, 28, 28), ((0, 0), (1, 1), (1, 1)))
    np_ = ((n + _BT - 1) // _BT) * _BT
    if np_ != n:
        xp = jnp.pad(xp, ((0, np_ - n), (0, 0), (0, 0)))
    b1 = _toeplitz1(w1f)
    b2 = _toeplitz2(w2f)
    s1 = jnp.tile(shift1, (1, 14))
    s2 = jnp.tile(shift2, (1, 7))
    out = pl.pallas_call(
        _fused_cnn_kernel,
        out_shape=jax.ShapeDtypeStruct((np_, 10), jnp.float32),
        grid=(np_ // _BT,),
        in_specs=[
            pl.BlockSpec((_BT, 30, 30), lambda i: (i, 0, 0)),
            pl.BlockSpec((90, 896), lambda i: (0, 0)),
            pl.BlockSpec((1, 448), lambda i: (0, 0)),
            pl.BlockSpec((1536, 896), lambda i: (0, 0)),
            pl.BlockSpec((1, 448), lambda i: (0, 0)),
            pl.BlockSpec((3584, 128), lambda i: (0, 0)),
            pl.BlockSpec((1, 128), lambda i: (0, 0)),
            pl.BlockSpec((128, 10), lambda i: (0, 0)),
            pl.BlockSpec((1, 10), lambda i: (0, 0)),
        ],
        out_specs=pl.BlockSpec((_BT, 10), lambda i: (i, 0)),
        compiler_params=pltpu.CompilerParams(
            dimension_semantics=("parallel",),
            vmem_limit_bytes=56 * 1024 * 1024),
    )(xp, b1, s1, b2, s2, fc1_w, fc1_b, fc2_w, fc2_b)
    return out[:n] if np_ != n else out


# speedup vs baseline: 19.2313x; 1.1288x over previous
import jax
import jax.numpy as jnp
from jax.experimental import pallas as pl
from jax.experimental.pallas import tpu as pltpu

_BT = 32


def _conv1_lhs(x):
    n = x.shape[0]
    xp = jnp.pad(x.reshape(n, 28, 28), ((0, 0), (1, 7), (1, 1)))
    pieces = []
    for j in range(4):
        for kh in range(3):
            r = j + kh
            pieces.append(jax.lax.slice(xp, (0, r, 0), (n, r + 29, 30), (1, 4, 1)))
    st = jnp.stack(pieces, axis=2)
    return st.reshape(n, 8, 4, 90).transpose(0, 2, 1, 3).reshape(n, 32, 90)


def _toeplitz1(w1f):
    xe = jnp.arange(0, 28, 2)
    xo = jnp.arange(1, 28, 2)
    xin = jnp.arange(30)

    def block(xs):
        d = xin[:, None] - xs[None, :]
        valid = (d >= 0) & (d <= 2)
        kh = jnp.arange(3)[:, None, None]
        tap = kh * 3 + jnp.clip(d, 0, 2)[None]
        t = w1f[tap, 0, :] * valid[None, :, :, None]
        z = jnp.zeros((3, 30, 1, 32), w1f.dtype)
        return jnp.concatenate([z, t, z], axis=2)

    return jnp.concatenate([block(xe), block(xo)], axis=2).reshape(90, 1024)


def _toeplitz2(w2f):
    xe = jnp.arange(0, 14, 2)
    xo = jnp.arange(1, 14, 2)
    xin = jnp.arange(16)

    def block(kh, xs):
        d = xin[:, None] - xs[None, :]
        valid = (d >= 0) & (d <= 2)
        tap = kh * 3 + jnp.clip(d, 0, 2)
        t = w2f[tap] * valid[:, :, None, None]
        t = t.transpose(0, 2, 1, 3)
        z = jnp.zeros((16, 32, 1, 64), w2f.dtype)
        return jnp.concatenate([t, z], axis=2)

    return [
        jnp.concatenate([block(kh, xe), block(kh, xo)], axis=2).reshape(512, 1024)
        for kh in range(3)
    ]


def _fused_cnn_kernel(l1_ref, b1_ref, s1_ref, bk0_ref, bk1_ref, bk2_ref,
                      s2_ref, fw1_ref, fb1_ref, fw2_ref, fb2_ref, o_ref):
    bsz = l1_ref.shape[0]
    lhs1 = l1_ref[...].reshape(bsz * 32, 90)
    c1 = jnp.dot(lhs1, b1_ref[...],
                 preferred_element_type=jnp.float32).reshape(bsz, 32, 1024)
    mx = jnp.maximum(c1[..., :512], c1[..., 512:])
    p1e = jnp.maximum(jnp.maximum(mx[:, 0:8], mx[:, 8:16]) + s1_ref[...], 0.0)
    p1o = jnp.maximum(jnp.maximum(mx[:, 16:24], mx[:, 24:32]) + s1_ref[...], 0.0)

    z1 = jnp.zeros((bsz, 1, 512), jnp.float32)
    re = jnp.concatenate([z1, p1o[:, 0:7]], axis=1).reshape(bsz * 8, 512)
    ro = jnp.concatenate([p1e[:, 0:7], z1], axis=1).reshape(bsz * 8, 512)
    re1 = jnp.concatenate([p1o[:, 0:7], z1], axis=1).reshape(bsz * 8, 512)
    ro1 = jnp.concatenate([p1e[:, 1:7], z1, z1], axis=1).reshape(bsz * 8, 512)

    f32 = jnp.float32
    c2e = (jnp.dot(re, bk0_ref[...], preferred_element_type=f32) +
           jnp.dot(ro, bk1_ref[...], preferred_element_type=f32) +
           jnp.dot(re1, bk2_ref[...], preferred_element_type=f32))
    c2o = (jnp.dot(ro, bk0_ref[...], preferred_element_type=f32) +
           jnp.dot(re1, bk1_ref[...], preferred_element_type=f32) +
           jnp.dot(ro1, bk2_ref[...], preferred_element_type=f32))
    m2 = jnp.maximum(c2e, c2o)
    m2 = jnp.maximum(m2[..., :512], m2[..., 512:])
    p2 = jnp.maximum(m2 + s2_ref[...], 0.0)

    flat = p2.reshape(bsz, 4096)
    h1 = jnp.dot(flat, fw1_ref[...],
                 preferred_element_type=f32) + fb1_ref[...]
    h1 = jnp.maximum(h1, 0.0)
    out = jnp.dot(h1, fw2_ref[...],
                  preferred_element_type=f32) + fb2_ref[...]
    o_ref[...] = out


def kernel(x, w1f, shift1, w2f, shift2, fc1_w, fc1_b, fc2_w, fc2_b):
    n = x.shape[0]
    lhs1 = _conv1_lhs(x)
    np_ = ((n + _BT - 1) // _BT) * _BT
    if np_ != n:
        lhs1 = jnp.pad(lhs1, ((0, np_ - n), (0, 0), (0, 0)))
    b1 = _toeplitz1(w1f)
    bk0, bk1, bk2 = _toeplitz2(w2f)
    zc = jnp.zeros((1, 32), shift1.dtype)
    s1 = jnp.concatenate([zc, jnp.tile(shift1, (1, 14)), zc],
                         axis=1).reshape(1, 1, 512)
    zc2 = jnp.zeros((1, 64), shift2.dtype)
    s2 = jnp.concatenate([jnp.tile(shift2, (1, 7)), zc2], axis=1)
    fw1 = jnp.concatenate([fc1_w, jnp.zeros((512, fc1_w.shape[1]), fc1_w.dtype)],
                          axis=0)
    out = pl.pallas_call(
        _fused_cnn_kernel,
        out_shape=jax.ShapeDtypeStruct((np_, 10), jnp.float32),
        grid=(np_ // _BT,),
        in_specs=[
            pl.BlockSpec((_BT, 32, 90), lambda i: (i, 0, 0)),
            pl.BlockSpec((90, 1024), lambda i: (0, 0)),
            pl.BlockSpec((1, 1, 512), lambda i: (0, 0, 0)),
            pl.BlockSpec((512, 1024), lambda i: (0, 0)),
            pl.BlockSpec((512, 1024), lambda i: (0, 0)),
            pl.BlockSpec((512, 1024), lambda i: (0, 0)),
            pl.BlockSpec((1, 512), lambda i: (0, 0)),
            pl.BlockSpec((4096, 128), lambda i: (0, 0)),
            pl.BlockSpec((1, 128), lambda i: (0, 0)),
            pl.BlockSpec((128, 10), lambda i: (0, 0)),
            pl.BlockSpec((1, 10), lambda i: (0, 0)),
        ],
        out_specs=pl.BlockSpec((_BT, 10), lambda i: (i, 0)),
        compiler_params=pltpu.CompilerParams(
            dimension_semantics=("parallel",),
            vmem_limit_bytes=56 * 1024 * 1024),
    )(lhs1, b1, s1, bk0, bk1, bk2, s2, fw1, fc1_b, fc2_w, fc2_b)
    return out[:n] if np_ != n else out


# speedup vs baseline: 20.5183x; 1.0669x over previous
import jax
import jax.numpy as jnp
from jax.experimental import pallas as pl
from jax.experimental.pallas import tpu as pltpu

_BT = 64


def _conv1_lhs(x):
    n = x.shape[0]
    xp = jnp.pad(x.reshape(n, 28, 28), ((0, 0), (1, 7), (1, 1)))
    pieces = []
    for j in range(4):
        for kh in range(3):
            r = j + kh
            pieces.append(jax.lax.slice(xp, (0, r, 0), (n, r + 29, 30), (1, 4, 1)))
    st = jnp.stack(pieces, axis=2)
    return st.reshape(n, 8, 4, 90).transpose(0, 2, 1, 3).reshape(n, 32, 90)


def _toeplitz1(w1f):
    xe = jnp.arange(0, 28, 2)
    xo = jnp.arange(1, 28, 2)
    xin = jnp.arange(30)

    def block(xs):
        d = xin[:, None] - xs[None, :]
        valid = (d >= 0) & (d <= 2)
        kh = jnp.arange(3)[:, None, None]
        tap = kh * 3 + jnp.clip(d, 0, 2)[None]
        t = w1f[tap, 0, :] * valid[None, :, :, None]
        z = jnp.zeros((3, 30, 1, 32), w1f.dtype)
        return jnp.concatenate([z, t, z], axis=2)

    return jnp.concatenate([block(xe), block(xo)], axis=2).reshape(90, 1024)


def _toeplitz2(w2f):
    xe = jnp.arange(0, 14, 2)
    xo = jnp.arange(1, 14, 2)
    xin = jnp.arange(16)

    def block(kh, xs):
        d = xin[:, None] - xs[None, :]
        valid = (d >= 0) & (d <= 2)
        tap = kh * 3 + jnp.clip(d, 0, 2)
        t = w2f[tap] * valid[:, :, None, None]
        t = t.transpose(0, 2, 1, 3)
        z = jnp.zeros((16, 32, 1, 64), w2f.dtype)
        return jnp.concatenate([t, z], axis=2)

    return [
        jnp.concatenate([block(kh, xe), block(kh, xo)], axis=2).reshape(512, 1024)
        for kh in range(3)
    ]


def _fused_cnn_kernel(l1_ref, b1_ref, s1_ref, bk0_ref, bk1_ref, bk2_ref,
                      s2_ref, fw1_ref, fb1_ref, fw2_ref, fb2_ref, o_ref):
    bsz = l1_ref.shape[0]
    bf16 = jnp.bfloat16
    lhs1 = l1_ref[...].reshape(bsz * 32, 90).astype(bf16)
    c1 = jnp.dot(lhs1, b1_ref[...],
                 preferred_element_type=jnp.float32).reshape(bsz, 32, 1024)
    mx = jnp.maximum(c1[..., :512], c1[..., 512:])
    p1e = jnp.maximum(jnp.maximum(mx[:, 0:8], mx[:, 8:16]) + s1_ref[...], 0.0)
    p1o = jnp.maximum(jnp.maximum(mx[:, 16:24], mx[:, 24:32]) + s1_ref[...], 0.0)

    z1 = jnp.zeros((bsz, 1, 512), jnp.float32)
    re = jnp.concatenate([z1, p1o[:, 0:7]], axis=1).reshape(bsz * 8, 512)
    ro = jnp.concatenate([p1e[:, 0:7], z1], axis=1).reshape(bsz * 8, 512)
    re1 = jnp.concatenate([p1o[:, 0:7], z1], axis=1).reshape(bsz * 8, 512)
    ro1 = jnp.concatenate([p1e[:, 1:7], z1, z1], axis=1).reshape(bsz * 8, 512)
    re = re.astype(bf16)
    ro = ro.astype(bf16)
    re1 = re1.astype(bf16)
    ro1 = ro1.astype(bf16)

    f32 = jnp.float32
    c2e = (jnp.dot(re, bk0_ref[...], preferred_element_type=f32) +
           jnp.dot(ro, bk1_ref[...], preferred_element_type=f32) +
           jnp.dot(re1, bk2_ref[...], preferred_element_type=f32))
    c2o = (jnp.dot(ro, bk0_ref[...], preferred_element_type=f32) +
           jnp.dot(re1, bk1_ref[...], preferred_element_type=f32) +
           jnp.dot(ro1, bk2_ref[...], preferred_element_type=f32))
    m2 = jnp.maximum(c2e, c2o)
    m2 = jnp.maximum(m2[..., :512], m2[..., 512:])
    p2 = jnp.maximum(m2 + s2_ref[...], 0.0)

    flat = p2.reshape(bsz, 4096).astype(bf16)
    h1 = jnp.dot(flat, fw1_ref[...],
                 preferred_element_type=f32) + fb1_ref[...]
    h1 = jnp.maximum(h1, 0.0).astype(bf16)
    out = jnp.dot(h1, fw2_ref[...],
                  preferred_element_type=f32) + fb2_ref[...]
    o_ref[...] = out


def kernel(x, w1f, shift1, w2f, shift2, fc1_w, fc1_b, fc2_w, fc2_b):
    n = x.shape[0]
    lhs1 = _conv1_lhs(x)
    np_ = ((n + _BT - 1) // _BT) * _BT
    if np_ != n:
        lhs1 = jnp.pad(lhs1, ((0, np_ - n), (0, 0), (0, 0)))
    bf16 = jnp.bfloat16
    b1 = _toeplitz1(w1f).astype(bf16)
    bk0, bk1, bk2 = (b.astype(bf16) for b in _toeplitz2(w2f))
    zc = jnp.zeros((1, 32), shift1.dtype)
    s1 = jnp.concatenate([zc, jnp.tile(shift1, (1, 14)), zc],
                         axis=1).reshape(1, 1, 512)
    zc2 = jnp.zeros((1, 64), shift2.dtype)
    s2 = jnp.concatenate([jnp.tile(shift2, (1, 7)), zc2], axis=1)
    fw1 = jnp.concatenate([fc1_w, jnp.zeros((512, fc1_w.shape[1]), fc1_w.dtype)],
                          axis=0).astype(bf16)
    fw2 = fc2_w.astype(bf16)
    out = pl.pallas_call(
        _fused_cnn_kernel,
        out_shape=jax.ShapeDtypeStruct((np_, 10), jnp.float32),
        grid=(np_ // _BT,),
        in_specs=[
            pl.BlockSpec((_BT, 32, 90), lambda i: (i, 0, 0)),
            pl.BlockSpec((90, 1024), lambda i: (0, 0)),
            pl.BlockSpec((1, 1, 512), lambda i: (0, 0, 0)),
            pl.BlockSpec((512, 1024), lambda i: (0, 0)),
            pl.BlockSpec((512, 1024), lambda i: (0, 0)),
            pl.BlockSpec((512, 1024), lambda i: (0, 0)),
            pl.BlockSpec((1, 512), lambda i: (0, 0)),
            pl.BlockSpec((4096, 128), lambda i: (0, 0)),
            pl.BlockSpec((1, 128), lambda i: (0, 0)),
            pl.BlockSpec((128, 10), lambda i: (0, 0)),
            pl.BlockSpec((1, 10), lambda i: (0, 0)),
        ],
        out_specs=pl.BlockSpec((_BT, 10), lambda i: (i, 0)),
        compiler_params=pltpu.CompilerParams(
            dimension_semantics=("parallel",),
            vmem_limit_bytes=56 * 1024 * 1024),
    )(lhs1, b1, s1, bk0, bk1, bk2, s2, fw1, fc1_b, fw2, fc2_b)
    return out[:n] if np_ != n else out


# speedup vs baseline: 22.5605x; 1.0995x over previous
import jax
import jax.numpy as jnp
from jax.experimental import pallas as pl
from jax.experimental.pallas import tpu as pltpu

_BT = 64


def _conv1_lhs(x):
    n = x.shape[0]
    xp = jnp.pad(x.reshape(n, 28, 28), ((0, 0), (1, 7), (1, 1)))
    pieces = []
    for j in range(4):
        for kh in range(3):
            r = j + kh
            pieces.append(jax.lax.slice(xp, (0, r, 0), (n, r + 29, 30), (1, 4, 1)))
    st = jnp.stack(pieces, axis=2)
    out = st.reshape(n, 8, 4, 90).transpose(0, 2, 1, 3).reshape(n, 32, 90)
    return out.astype(jnp.bfloat16)


def _toeplitz1(w1f):
    xe = jnp.arange(0, 28, 2)
    xo = jnp.arange(1, 28, 2)
    xin = jnp.arange(30)

    def block(xs):
        d = xin[:, None] - xs[None, :]
        valid = (d >= 0) & (d <= 2)
        kh = jnp.arange(3)[:, None, None]
        tap = kh * 3 + jnp.clip(d, 0, 2)[None]
        t = w1f[tap, 0, :] * valid[None, :, :, None]
        z = jnp.zeros((3, 30, 1, 32), w1f.dtype)
        return jnp.concatenate([z, t, z], axis=2)

    return jnp.concatenate([block(xe), block(xo)], axis=2).reshape(90, 1024)


def _toeplitz2(w2f):
    xe = jnp.arange(0, 14, 2)
    xo = jnp.arange(1, 14, 2)
    xin = jnp.arange(16)

    def block(kh, xs):
        d = xin[:, None] - xs[None, :]
        valid = (d >= 0) & (d <= 2)
        tap = kh * 3 + jnp.clip(d, 0, 2)
        t = w2f[tap] * valid[:, :, None, None]
        t = t.transpose(0, 2, 1, 3)
        z = jnp.zeros((16, 32, 1, 64), w2f.dtype)
        return jnp.concatenate([t, z], axis=2)

    return jnp.concatenate([
        jnp.concatenate([block(kh, xe), block(kh, xo)], axis=2).reshape(512, 1024)
        for kh in range(3)
    ], axis=0)


def _fused_cnn_kernel(l1_ref, b1_ref, s1_ref, b2_ref,
                      s2_ref, fw1_ref, fb1_ref, fw2_ref, fb2_ref, o_ref):
    bsz = l1_ref.shape[0]
    bf16 = jnp.bfloat16
    lhs1 = l1_ref[...].reshape(bsz * 32, 90)
    c1 = jnp.dot(lhs1, b1_ref[...],
                 preferred_element_type=jnp.float32).reshape(bsz, 32, 1024)
    mx = jnp.maximum(c1[..., :512], c1[..., 512:])
    p1e = jnp.maximum(jnp.maximum(mx[:, 0:8], mx[:, 8:16]) + s1_ref[...], 0.0)
    p1o = jnp.maximum(jnp.maximum(mx[:, 16:24], mx[:, 24:32]) + s1_ref[...], 0.0)

    pe = p1e.astype(bf16)
    po = p1o.astype(bf16)
    z1 = jnp.zeros((bsz, 1, 512), bf16)
    re = jnp.concatenate([z1, po[:, 0:7]], axis=1)
    ro = jnp.concatenate([pe[:, 0:7], z1], axis=1)
    re1 = jnp.concatenate([po[:, 0:7], z1], axis=1)
    ro1 = jnp.concatenate([pe[:, 1:7], z1, z1], axis=1)

    f32 = jnp.float32
    l2e = jnp.concatenate([re, ro, re1], axis=-1)
    l2o = jnp.concatenate([ro, re1, ro1], axis=-1)
    lhs2 = jnp.concatenate([l2e, l2o], axis=1).reshape(bsz * 16, 1536)
    c2 = jnp.dot(lhs2, b2_ref[...],
                 preferred_element_type=f32).reshape(bsz, 16, 1024)
    m2 = jnp.maximum(c2[:, 0:8], c2[:, 8:16])
    m2 = jnp.maximum(m2[..., :512], m2[..., 512:])
    p2 = jnp.maximum(m2 + s2_ref[...], 0.0)

    flat = p2.reshape(bsz, 4096).astype(bf16)
    h1 = jnp.dot(flat, fw1_ref[...],
                 preferred_element_type=f32) + fb1_ref[...]
    h1 = jnp.maximum(h1, 0.0).astype(bf16)
    out = jnp.dot(h1, fw2_ref[...],
                  preferred_element_type=f32) + fb2_ref[...]
    o_ref[...] = out


def kernel(x, w1f, shift1, w2f, shift2, fc1_w, fc1_b, fc2_w, fc2_b):
    n = x.shape[0]
    lhs1 = _conv1_lhs(x)
    np_ = ((n + _BT - 1) // _BT) * _BT
    if np_ != n:
        lhs1 = jnp.pad(lhs1, ((0, np_ - n), (0, 0), (0, 0)))
    bf16 = jnp.bfloat16
    b1 = _toeplitz1(w1f).astype(bf16)
    b2 = _toeplitz2(w2f).astype(bf16)
    zc = jnp.zeros((1, 32), shift1.dtype)
    s1 = jnp.concatenate([zc, jnp.tile(shift1, (1, 14)), zc],
                         axis=1).reshape(1, 1, 512)
    zc2 = jnp.zeros((1, 64), shift2.dtype)
    s2 = jnp.concatenate([jnp.tile(shift2, (1, 7)), zc2],
                         axis=1).reshape(1, 1, 512)
    fw1 = jnp.concatenate([fc1_w, jnp.zeros((512, fc1_w.shape[1]), fc1_w.dtype)],
                          axis=0).astype(bf16)
    fw2 = fc2_w.astype(bf16)
    out = pl.pallas_call(
        _fused_cnn_kernel,
        out_shape=jax.ShapeDtypeStruct((np_, 10), jnp.float32),
        grid=(np_ // _BT,),
        in_specs=[
            pl.BlockSpec((_BT, 32, 90), lambda i: (i, 0, 0)),
            pl.BlockSpec((90, 1024), lambda i: (0, 0)),
            pl.BlockSpec((1, 1, 512), lambda i: (0, 0, 0)),
            pl.BlockSpec((1536, 1024), lambda i: (0, 0)),
            pl.BlockSpec((1, 1, 512), lambda i: (0, 0, 0)),
            pl.BlockSpec((4096, 128), lambda i: (0, 0)),
            pl.BlockSpec((1, 128), lambda i: (0, 0)),
            pl.BlockSpec((128, 10), lambda i: (0, 0)),
            pl.BlockSpec((1, 10), lambda i: (0, 0)),
        ],
        out_specs=pl.BlockSpec((_BT, 10), lambda i: (i, 0)),
        compiler_params=pltpu.CompilerParams(
            dimension_semantics=("parallel",),
            vmem_limit_bytes=56 * 1024 * 1024),
    )(lhs1, b1, s1, b2, s2, fw1, fc1_b, fw2, fc2_b)
    return out[:n] if np_ != n else out
